# baseline (speedup 1.0000x reference)
import functools

import jax
import jax.numpy as jnp
from jax import lax
from jax.experimental import pallas as pl
from jax.experimental.pallas import tpu as pltpu
from jax.experimental.pallas import tpu_sc as plsc

D_MODEL = 64
NUM_CORES = 2
NUM_SUBCORES = 16
NUM_WORKERS = NUM_CORES * NUM_SUBCORES
R = 4


@functools.cache
def _make_embed(n_rows: int, n_cols: int):
    assert n_rows % (NUM_WORKERS * 2 * R) == 0
    rows_per_worker = n_rows // NUM_WORKERS
    n_groups = rows_per_worker // R
    splits = [(o, min(128, n_cols - o)) for o in range(0, n_cols, 128)]
    mesh = plsc.VectorSubcoreMesh(core_axis_name="c", subcore_axis_name="s")

    @functools.partial(
        pl.kernel,
        out_type=jax.ShapeDtypeStruct((n_rows, n_cols, D_MODEL), jnp.float32),
        mesh=mesh,
        scratch_types=[
            pltpu.VMEM((2, R, n_cols), jnp.int32),
            pltpu.VMEM((2, R, n_cols, D_MODEL), jnp.float32),
            [pltpu.SemaphoreType.DMA] * 2,
            [pltpu.SemaphoreType.DMA] * 2,
            [pltpu.SemaphoreType.DMA] * 2,
        ],
        compiler_params=pltpu.CompilerParams(use_tc_tiling_on_sc=False),
    )
    def embed(idx_hbm, table_hbm, out_hbm, idx_v, rows_v, isem, gsem, ssem):
        wid = lax.axis_index("s") * NUM_CORES + lax.axis_index("c")
        base = wid * rows_per_worker

        def idx_copy(g, b):
            row0 = pl.multiple_of(base + g * R, R)
            return pltpu.make_async_copy(
                idx_hbm.at[pl.ds(row0, R), :], idx_v.at[b], isem[b])

        def gather_copies(b):
            for r in range(R):
                for o, w in splits:
                    yield pltpu.make_async_copy(
                        table_hbm.at[idx_v.at[b, r, pl.ds(o, w)]],
                        rows_v.at[b, r, pl.ds(o, w), :],
                        gsem[b],
                    )

        def store_copy(g, b):
            row0 = pl.multiple_of(base + g * R, R)
            return pltpu.make_async_copy(
                rows_v.at[b], out_hbm.at[pl.ds(row0, R)], ssem[b])

        idx_copy(0, 0).start()
        idx_copy(0, 0).wait()

        @pl.loop(0, n_groups, step=2)
        def sup(t):
            for b in (0, 1):
                g = t + b

                @pl.when(g >= 1)
                def _():
                    idx_copy(g, b).wait()

                @pl.when(g >= 2)
                def _():
                    store_copy(g - 2, b).wait()

                for c in gather_copies(b):
                    c.start()

                @pl.when(g >= 1)
                def _():
                    for c in gather_copies(1 - b):
                        c.wait()
                    store_copy(g - 1, 1 - b).start()

                @pl.when(g + 1 < n_groups)
                def _():
                    idx_copy(g + 1, 1 - b).start()

        for c in gather_copies(1):
            c.wait()
        store_copy(n_groups - 1, 1).start()
        store_copy(n_groups - 2, 0).wait()
        store_copy(n_groups - 1, 1).wait()

    return embed


def kernel(x, table):
    return _make_embed(*x.shape)(x, table)


# speedup vs baseline: 1.6377x; 1.6377x over previous
import functools

import jax
import jax.numpy as jnp
from jax import lax
from jax.experimental import pallas as pl
from jax.experimental.pallas import tpu as pltpu
from jax.experimental.pallas import tpu_sc as plsc

D_MODEL = 64
D_PAD = 128
NUM_CORES = 2
NUM_SUBCORES = 16
NUM_WORKERS = NUM_CORES * NUM_SUBCORES
SG = 256
CHUNK = 1024


@functools.cache
def _make_embed(B: int):
    assert B % (NUM_WORKERS * 2 * CHUNK) == 0
    per_worker = B // NUM_WORKERS
    n_chunks = per_worker // CHUNK
    mesh = plsc.VectorSubcoreMesh(core_axis_name="c", subcore_axis_name="s")

    @functools.partial(
        pl.kernel,
        out_type=jax.ShapeDtypeStruct((B, D_PAD), jnp.float32),
        mesh=mesh,
        scratch_types=[
            pltpu.VMEM((2, CHUNK // 128, 128), jnp.int32),
            pltpu.VMEM((2, SG, D_MODEL), jnp.float32),
            [pltpu.SemaphoreType.DMA] * 2,
            [pltpu.SemaphoreType.DMA] * 2,
            [pltpu.SemaphoreType.DMA] * 2,
        ],
        compiler_params=pltpu.CompilerParams(use_tc_tiling_on_sc=False),
    )
    def embed(idx_hbm, table_hbm, out_hbm, idx_v, rows_v, isem, gsem, ssem):
        wid = lax.axis_index("s") * NUM_CORES + lax.axis_index("c")
        base = wid * per_worker

        def idx_copy(c, cb):
            row0 = pl.multiple_of((base + c * CHUNK) // 128, 8)
            return pltpu.make_async_copy(
                idx_hbm.at[pl.ds(row0, CHUNK // 128), :], idx_v.at[cb], isem[cb])

        def gather_copy(cb, q, j, rs):
            return pltpu.make_async_copy(
                table_hbm.at[idx_v.at[cb, 2 * q + j]],
                rows_v.at[rs, pl.ds(j * 128, 128), :],
                gsem[rs],
            )

        def store_copy(sg, rs):
            off = pl.multiple_of(base + sg * SG, SG)
            return pltpu.make_async_copy(
                rows_v.at[rs],
                out_hbm.at[pl.ds(off, SG), pl.ds(0, D_MODEL)], ssem[rs])

        idx_copy(0, 0).start()
        idx_copy(1, 1).start()
        idx_copy(0, 0).wait()

        @pl.loop(0, n_chunks, step=2)
        def sup(t):
            for cb in (0, 1):
                c = t + cb

                @pl.when(c >= 1)
                def _():
                    idx_copy(c, cb).wait()

                for q in range(4):
                    sg = 4 * c + q
                    rs = q % 2

                    @pl.when(sg >= 2)
                    def _():
                        store_copy(sg - 2, rs).wait()

                    for j in range(2):
                        gather_copy(cb, q, j, rs).start()

                    @pl.when(sg >= 1)
                    def _():
                        for j in range(2):
                            gather_copy(1 - cb if q == 0 else cb,
                                        3 if q == 0 else q - 1, j, 1 - rs).wait()
                        store_copy(sg - 1, 1 - rs).start()

                    if q == 0:
                        @pl.when((c >= 1) & (c + 1 < n_chunks))
                        def _():
                            idx_copy(c + 1, 1 - cb).start()

        last = 4 * n_chunks - 1
        for j in range(2):
            gather_copy(1, 3, j, 1).wait()
        store_copy(last, 1).start()
        store_copy(last - 1, 0).wait()
        store_copy(last, 1).wait()

    return embed


def kernel(x, table):
    B = x.size
    idx2 = x.reshape(B // 128, 128)
    out2d = _make_embed(B)(idx2, table)
    return out2d[:, :D_MODEL].reshape(x.shape + (D_MODEL,))


# speedup vs baseline: 1.6407x; 1.0019x over previous
import functools

import jax
import jax.numpy as jnp
from jax import lax
from jax.experimental import pallas as pl
from jax.experimental.pallas import tpu as pltpu
from jax.experimental.pallas import tpu_sc as plsc

D_MODEL = 64
D_PAD = 128
NUM_CORES = 2
NUM_SUBCORES = 16
NUM_WORKERS = NUM_CORES * NUM_SUBCORES
SG = 512
CHUNK = 2048
SPC = CHUNK // SG
JG = SG // 128


@functools.cache
def _make_embed(B: int):
    assert B % (NUM_WORKERS * 2 * CHUNK) == 0
    per_worker = B // NUM_WORKERS
    n_chunks = per_worker // CHUNK
    mesh = plsc.VectorSubcoreMesh(core_axis_name="c", subcore_axis_name="s")

    @functools.partial(
        pl.kernel,
        out_type=jax.ShapeDtypeStruct((B, D_PAD), jnp.float32),
        mesh=mesh,
        scratch_types=[
            pltpu.VMEM((2, CHUNK // 128, 128), jnp.int32),
            pltpu.VMEM((2, SG, D_MODEL), jnp.float32),
            [pltpu.SemaphoreType.DMA] * 2,
            [pltpu.SemaphoreType.DMA] * 2,
            [pltpu.SemaphoreType.DMA] * 2,
        ],
        compiler_params=pltpu.CompilerParams(use_tc_tiling_on_sc=False),
    )
    def embed(idx_hbm, table_hbm, out_hbm, idx_v, rows_v, isem, gsem, ssem):
        wid = lax.axis_index("s") * NUM_CORES + lax.axis_index("c")
        base = wid * per_worker

        def idx_copy(c, cb):
            row0 = pl.multiple_of((base + c * CHUNK) // 128, 8)
            return pltpu.make_async_copy(
                idx_hbm.at[pl.ds(row0, CHUNK // 128), :], idx_v.at[cb], isem[cb])

        def gather_copy(cb, q, j, rs):
            return pltpu.make_async_copy(
                table_hbm.at[idx_v.at[cb, JG * q + j]],
                rows_v.at[rs, pl.ds(j * 128, 128), :],
                gsem[rs],
            )

        def store_copy(sg, rs):
            off = pl.multiple_of(base + sg * SG, SG)
            return pltpu.make_async_copy(
                rows_v.at[rs],
                out_hbm.at[pl.ds(off, SG), pl.ds(0, D_MODEL)], ssem[rs])

        idx_copy(0, 0).start()
        idx_copy(1, 1).start()
        idx_copy(0, 0).wait()

        @pl.loop(0, n_chunks, step=2)
        def sup(t):
            for cb in (0, 1):
                c = t + cb

                @pl.when(c >= 1)
                def _():
                    idx_copy(c, cb).wait()

                for q in range(SPC):
                    sg = SPC * c + q
                    rs = q % 2

                    @pl.when(sg >= 2)
                    def _():
                        store_copy(sg - 2, rs).wait()

                    for j in range(JG):
                        gather_copy(cb, q, j, rs).start()

                    @pl.when(sg >= 1)
                    def _():
                        for j in range(JG):
                            gather_copy(1 - cb if q == 0 else cb,
                                        SPC - 1 if q == 0 else q - 1, j, 1 - rs).wait()
                        store_copy(sg - 1, 1 - rs).start()

                    if q == 0:
                        @pl.when((c >= 1) & (c + 1 < n_chunks))
                        def _():
                            idx_copy(c + 1, 1 - cb).start()

        last = SPC * n_chunks - 1
        for j in range(JG):
            gather_copy(1, SPC - 1, j, 1).wait()
        store_copy(last, 1).start()
        store_copy(last - 1, 0).wait()
        store_copy(last, 1).wait()

    return embed


def kernel(x, table):
    B = x.size
    idx2 = x.reshape(B // 128, 128)
    out2d = _make_embed(B)(idx2, table)
    return out2d[:, :D_MODEL].reshape(x.shape + (D_MODEL,))


# speedup vs baseline: 1.6460x; 1.0032x over previous
import functools

import jax
import jax.numpy as jnp
from jax import lax
from jax.experimental import pallas as pl
from jax.experimental.pallas import tpu as pltpu
from jax.experimental.pallas import tpu_sc as plsc

D_MODEL = 64
D_PAD = 128
NUM_CORES = 2
NUM_SUBCORES = 16
NUM_WORKERS = NUM_CORES * NUM_SUBCORES
SG = 512
CHUNK = 2048
SPC = CHUNK // SG
JG = SG // 128


@functools.cache
def _make_embed(B: int):
    assert B % (NUM_WORKERS * 2 * CHUNK) == 0
    assert SPC % 2 == 0
    per_worker = B // NUM_WORKERS
    n_chunks = per_worker // CHUNK
    mesh = plsc.VectorSubcoreMesh(core_axis_name="c", subcore_axis_name="s")

    @functools.partial(
        pl.kernel,
        out_type=jax.ShapeDtypeStruct((B, D_PAD), jnp.float32),
        mesh=mesh,
        scratch_types=[
            pltpu.VMEM((2, CHUNK // 128, 128), jnp.int32),
            pltpu.VMEM((2, SG, D_MODEL), jnp.float32),
            [pltpu.SemaphoreType.DMA] * 2,
            [pltpu.SemaphoreType.DMA] * 2,
            [pltpu.SemaphoreType.DMA] * 2,
        ],
        compiler_params=pltpu.CompilerParams(use_tc_tiling_on_sc=False),
    )
    def embed(idx_hbm, table_hbm, out_hbm, idx_v, rows_v, isem, gsem, ssem):
        wid = lax.axis_index("s") * NUM_CORES + lax.axis_index("c")
        base = wid * per_worker

        def idx_copy(c, cb):
            row0 = pl.multiple_of((base + c * CHUNK) // 128, 8)
            return pltpu.make_async_copy(
                idx_hbm.at[pl.ds(row0, CHUNK // 128), :], idx_v.at[cb], isem[cb])

        def gather_copy(cb, q, j, rs):
            return pltpu.make_async_copy(
                table_hbm.at[idx_v.at[cb, JG * q + j]],
                rows_v.at[rs, pl.ds(j * 128, 128), :],
                gsem[rs],
            )

        def store_copy(sg, rs):
            off = pl.multiple_of(base + sg * SG, SG)
            return pltpu.make_async_copy(
                rows_v.at[rs],
                out_hbm.at[pl.ds(off, SG), pl.ds(0, D_MODEL)], ssem[rs])

        idx_copy(0, 0).start()
        idx_copy(1, 1).start()
        idx_copy(0, 0).wait()

        @pl.loop(0, n_chunks, step=2)
        def sup(t):
            for cb in (0, 1):
                c = t + cb

                @pl.when(c >= 1)
                def _():
                    idx_copy(c, cb).wait()

                for q in range(SPC):
                    sg = SPC * c + q
                    rs = q % 2

                    @pl.when(sg >= 2)
                    def _():
                        store_copy(sg - 2, rs).wait()

                    for j in range(JG):
                        gather_copy(cb, q, j, rs).start()

                    @pl.when(sg >= 1)
                    def _():
                        for j in range(JG):
                            gather_copy(1 - cb if q == 0 else cb,
                                        SPC - 1 if q == 0 else q - 1, j, 1 - rs).wait()
                        store_copy(sg - 1, 1 - rs).start()

                    if q == 0:
                        @pl.when((c >= 1) & (c + 1 < n_chunks))
                        def _():
                            idx_copy(c + 1, 1 - cb).start()

        last = SPC * n_chunks - 1
        for j in range(JG):
            gather_copy(1, SPC - 1, j, 1).wait()
        store_copy(last, 1).start()
        store_copy(last - 1, 0).wait()
        store_copy(last, 1).wait()

    return embed


def kernel(x, table):
    B = x.size
    idx2 = x.reshape(B // 128, 128)
    out2d = _make_embed(B)(idx2, table)
    return out2d[:, :D_MODEL].reshape(x.shape + (D_MODEL,))
